# baseline (speedup 1.0000x reference)
import functools

import jax
import jax.numpy as jnp
import numpy as np
from jax.experimental import pallas as pl
from jax.sharding import Mesh, PartitionSpec as P


def _gcn_layer_kernel(adj_ref, v_ref, w_ref, b_ref, out_ref, *, relu):
    a16 = adj_ref[...].astype(jnp.bfloat16)
    t = jnp.dot(a16, v_ref[...], preferred_element_type=jnp.float32)
    t = jnp.dot(t.astype(jnp.bfloat16), w_ref[...].astype(jnp.bfloat16),
                preferred_element_type=jnp.float32) + b_ref[...]
    if relu:
        t = jnp.maximum(t, 0.0)
    out_ref[...] = t.astype(out_ref.dtype)


def _gcn_layer(adj, v, w, b, *, relu, out_dtype, bm):
    n, k = adj.shape
    d = w.shape[1]
    grid = (n // bm,)
    return pl.pallas_call(
        functools.partial(_gcn_layer_kernel, relu=relu),
        grid=grid,
        in_specs=[
            pl.BlockSpec((bm, k), lambda i: (i, 0)),
            pl.BlockSpec((k, v.shape[1]), lambda i: (0, 0)),
            pl.BlockSpec(w.shape, lambda i: (0, 0)),
            pl.BlockSpec((1, d), lambda i: (0, 0)),
        ],
        out_specs=pl.BlockSpec((bm, d), lambda i: (i, 0)),
        out_shape=jax.ShapeDtypeStruct((n, d), out_dtype),
    )(adj, v, w, b.reshape(1, d))


def _gcn_both_layers(adj_s, x16, W1, b1, W2, b2, *, bm):
    h_local = _gcn_layer(adj_s, x16, W1, b1, relu=True,
                         out_dtype=jnp.bfloat16, bm=bm)
    h_full = jax.lax.all_gather(h_local, "i", axis=0, tiled=True)
    return _gcn_layer(adj_s, h_full, W2, b2, relu=False,
                      out_dtype=jnp.float32, bm=bm)


def kernel(x, adj, W1, b1, W2, b2):
    x16 = x.astype(jnp.bfloat16)
    devs = jax.devices()
    n_dev = 2 if (len(devs) >= 2 and adj.shape[0] % 2 == 0) else 1
    if n_dev == 1:
        h16 = _gcn_layer(adj, x16, W1, b1, relu=True, out_dtype=jnp.bfloat16, bm=400)
        return _gcn_layer(adj, h16, W2, b2, relu=False, out_dtype=jnp.float32, bm=400)
    mesh = Mesh(np.array(devs[:n_dev]), ("i",))
    fn = jax.shard_map(
        functools.partial(_gcn_both_layers, bm=200),
        mesh=mesh,
        in_specs=(P("i", None), P(None, None), P(None, None), P(None),
                  P(None, None), P(None)),
        out_specs=P("i", None),
        check_vma=False,
    )
    return fn(adj, x16, W1, b1, W2, b2)


# speedup vs baseline: 3.2472x; 3.2472x over previous
import functools

import jax
import jax.numpy as jnp
import numpy as np
from jax.experimental import pallas as pl
from jax.sharding import Mesh, PartitionSpec as P


def _gcn_layer_kernel(adj_ref, v_ref, w_ref, b_ref, out_ref, *, relu):
    a16 = adj_ref[...].astype(jnp.bfloat16)
    t = jnp.dot(a16, v_ref[...], preferred_element_type=jnp.float32)
    t = jnp.dot(t.astype(jnp.bfloat16), w_ref[...].astype(jnp.bfloat16),
                preferred_element_type=jnp.float32) + b_ref[...]
    if relu:
        t = jnp.maximum(t, 0.0)
    out_ref[...] = t.astype(out_ref.dtype)


def _gcn_layer(adj, v, w, b, *, relu, out_dtype, bm):
    n, k = adj.shape
    d = w.shape[1]
    grid = (n // bm,)
    return pl.pallas_call(
        functools.partial(_gcn_layer_kernel, relu=relu),
        grid=grid,
        in_specs=[
            pl.BlockSpec((bm, k), lambda i: (i, 0)),
            pl.BlockSpec((k, v.shape[1]), lambda i: (0, 0)),
            pl.BlockSpec(w.shape, lambda i: (0, 0)),
            pl.BlockSpec((1, d), lambda i: (0, 0)),
        ],
        out_specs=pl.BlockSpec((bm, d), lambda i: (i, 0)),
        out_shape=jax.ShapeDtypeStruct((n, d), out_dtype),
    )(adj, v, w, b.reshape(1, d))


def _gcn_both_layers(adj_s, x16, W1, b1, W2, b2, *, bm):
    h_local = _gcn_layer(adj_s, x16, W1, b1, relu=True,
                         out_dtype=jnp.bfloat16, bm=bm)
    h_full = jax.lax.all_gather(h_local, "i", axis=0, tiled=True)
    return _gcn_layer(adj_s, h_full, W2, b2, relu=False,
                      out_dtype=jnp.float32, bm=bm)


def kernel(x, adj, W1, b1, W2, b2):
    x16 = x.astype(jnp.bfloat16)
    h16 = _gcn_layer(adj, x16, W1, b1, relu=True, out_dtype=jnp.bfloat16, bm=400)
    return _gcn_layer(adj, h16, W2, b2, relu=False, out_dtype=jnp.float32, bm=400)
